# baseline (speedup 1.0000x reference)
import jax
import jax.numpy as jnp
from jax import lax
from jax.experimental import pallas as pl
from jax.experimental.pallas import tpu as pltpu
from jax.experimental.pallas import tpu_sc as plsc

N_NODES = 10000
N_EDGES = 320000
N_GRAPHS = 100
D_FEAT = 128
HID = 16
N_MIC = 10
N_ACTS = 8

NC = 2
NS = 16
NW = NC * NS
CHA = 256
ROWS_A = 40
N_ROWS = NW * ROWS_A
E_PAD = N_ROWS * CHA
NPAD = 10240
NPK = NPAD // 16
NPK_X = N_NODES // 16
G_PAD = 128
POOL_CH = NPAD // NW

_HIGH = lax.Precision.HIGHEST
NBUF = 4


def _sc_deg_body(eip, zeros1, ones1, deg_out, idx_v, ones_v, sh_deg, dsem):
    c = lax.axis_index("c")
    s = lax.axis_index("s")

    @pl.when(s == 0)
    def _init():
        pltpu.sync_copy(zeros1, sh_deg)

    pltpu.sync_copy(ones1, ones_v)
    wid = s * NC + c
    pltpu.sync_copy(eip.at[1].at[pl.ds(wid * ROWS_A, ROWS_A)], idx_v)
    plsc.subcore_barrier()

    def group(g, carry):
        descs = [
            pltpu.async_copy(ones_v, sh_deg.at[idx_v.at[4 * g + b]], dsem,
                             add=True)
            for b in range(4)
        ]
        for d in descs:
            d.wait()
        return carry

    lax.fori_loop(0, ROWS_A // 4, group, 0)
    plsc.subcore_barrier()

    @pl.when(s == 0)
    def _flush():
        pltpu.sync_copy(sh_deg, deg_out.at[c])


def _sc_agg_body(eia, xn, zeros16, agg_out,
                 sidx, didx, bufs, sh_acc, xn_sh, gsems, ssems):
    c = lax.axis_index("c")
    s = lax.axis_index("s")

    @pl.when(s == 0)
    def _init():
        pltpu.sync_copy(zeros16, sh_acc)
        pltpu.sync_copy(xn, xn_sh)

    wid = s * NC + c
    pltpu.sync_copy(eia.at[0].at[pl.ds(wid * ROWS_A, ROWS_A)], sidx)
    pltpu.sync_copy(eia.at[1].at[pl.ds(wid * ROWS_A, ROWS_A)], didx)
    plsc.subcore_barrier()

    for b in range(NBUF):
        pltpu.async_copy(xn_sh.at[sidx.at[b]], bufs[b], gsems[b])

    def step(t, carry):
        j = NBUF * t
        for b in range(NBUF):
            pltpu.make_async_copy(xn_sh.at[sidx.at[j + b]], bufs[b],
                                  gsems[b]).wait()
            pltpu.async_copy(bufs[b], sh_acc.at[didx.at[j + b]], ssems[b],
                             add=True)
        for b in range(NBUF):
            @pl.when(j + b + NBUF < ROWS_A)
            def _next(b=b):
                pltpu.make_async_copy(bufs[b], sh_acc.at[didx.at[j + b]],
                                      ssems[b]).wait()
                pltpu.async_copy(xn_sh.at[sidx.at[j + b + NBUF]], bufs[b],
                                 gsems[b])
        return carry

    lax.fori_loop(0, ROWS_A // NBUF, step, 0)
    for b in range(NBUF):
        pltpu.make_async_copy(bufs[b],
                              sh_acc.at[didx.at[ROWS_A - NBUF + b]],
                              ssems[b]).wait()
    plsc.subcore_barrier()

    @pl.when(s == 0)
    def _flush():
        pltpu.sync_copy(sh_acc, agg_out.at[c])


def _sc_pool_body(h, batch3, ones16, zerosg, sums_out, cnt_out,
                  bidx, h_v, ones_v, sh_sums, sh_cnt, psem):
    c = lax.axis_index("c")
    s = lax.axis_index("s")

    @pl.when(s == 0)
    def _init():
        pltpu.sync_copy(zerosg, sh_sums)
        pltpu.sync_copy(zerosg, sh_cnt)

    wid = s * NC + c
    pltpu.sync_copy(batch3.at[wid], bidx)
    pltpu.sync_copy(h.at[pl.ds(wid * POOL_CH, POOL_CH)], h_v)
    pltpu.sync_copy(ones16, ones_v)
    plsc.subcore_barrier()

    d1 = pltpu.async_copy(h_v, sh_sums.at[bidx], psem, add=True)
    d2 = pltpu.async_copy(ones_v, sh_cnt.at[bidx], psem, add=True)
    d1.wait()
    d2.wait()
    plsc.subcore_barrier()

    @pl.when(s == 0)
    def _flush():
        pltpu.sync_copy(sh_sums, sums_out.at[c])
        pltpu.sync_copy(sh_cnt, cnt_out.at[c])


def _tc_xw_body(x3_ref, w_ref, xw_ref):
    cols = []
    for k in range(16):
        xk = x3_ref[:, k, :]
        cols.append(jnp.dot(xk, w_ref[...],
                            preferred_element_type=jnp.float32,
                            precision=_HIGH))
    xw_p = jnp.concatenate(cols, axis=1)
    xw_ref[...] = jnp.concatenate(
        [xw_p, jnp.zeros((NPK - NPK_X, 16 * HID), jnp.float32)], axis=0)


def _tc_xn_body(xw_ref, degp_ref, bmat_ref, xn_ref, dinv_ref):
    deg2 = degp_ref[0] + degp_ref[1]
    deg_p = jnp.dot(deg2, bmat_ref[...],
                    preferred_element_type=jnp.float32,
                    precision=_HIGH) + 1.0
    dinv_p = lax.rsqrt(deg_p)
    dinv_ref[...] = dinv_p
    xn_ref[...] = xw_ref[...] * dinv_p


def _tc_h_body(aggp_ref, xn_ref, dinv_ref, bconv_ref, h_ref):
    agg = aggp_ref[0] + aggp_ref[1] + xn_ref[...]
    h_ref[...] = jnp.maximum(dinv_ref[...] * agg + bconv_ref[...], 0.0)


def _elu(v):
    return jnp.where(v > 0.0, v, jnp.exp(jnp.minimum(v, 0.0)) - 1.0)


def _tc_head_body(sums_ref, cnt_ref, w1_ref, b1_ref, w2_ref, b2_ref,
                  wout_ref, bout_ref, out_ref):
    sums = sums_ref[0] + sums_ref[1]
    cnt = cnt_ref[0] + cnt_ref[1]
    g = sums / jnp.maximum(cnt, 1.0)
    g = _elu(jnp.dot(g, w1_ref[...],
                     preferred_element_type=jnp.float32, precision=_HIGH)
             + b1_ref[...])
    g = _elu(jnp.dot(g, w2_ref[...],
                     preferred_element_type=jnp.float32, precision=_HIGH)
             + b2_ref[...])
    out_ref[...] = jnp.dot(g, wout_ref[...],
                           preferred_element_type=jnp.float32,
                           precision=_HIGH) + bout_ref[...]


def kernel(x, edge_index, batch, W_conv, b_conv, W1, b1, W2, b2, W_out, b_out):
    f32 = jnp.float32
    eip = jnp.pad(edge_index.astype(jnp.int32),
                  ((0, 0), (0, E_PAD - N_EDGES)),
                  constant_values=N_NODES).reshape(2, N_ROWS, CHA)
    zeros16 = jnp.zeros((NPAD, HID), f32)
    zeros1 = jnp.zeros((NPAD,), f32)
    ones1 = jnp.ones((CHA,), f32)
    ones16 = jnp.ones((POOL_CH, HID), f32)
    zerosg = jnp.zeros((G_PAD, HID), f32)
    bmat = jnp.repeat(jnp.eye(16, dtype=f32), 16, axis=1)
    batch3 = jnp.pad(batch.astype(jnp.int32), (0, NPAD - N_NODES),
                     constant_values=G_PAD - 1).reshape(NW, POOL_CH)

    mesh = plsc.VectorSubcoreMesh(core_axis_name="c", subcore_axis_name="s",
                                  num_cores=NC, num_subcores=NS)
    sc_params = pltpu.CompilerParams(use_tc_tiling_on_sc=False)

    deg_parts = pl.kernel(
        _sc_deg_body,
        out_type=jax.ShapeDtypeStruct((NC, NPAD), f32),
        mesh=mesh,
        scratch_types=[
            pltpu.VMEM((ROWS_A, CHA), jnp.int32),
            pltpu.VMEM((CHA,), f32),
            pltpu.VMEM_SHARED((NPAD,), f32),
            pltpu.SemaphoreType.DMA,
        ],
        compiler_params=sc_params,
    )(eip, zeros1, ones1)

    xw_p = pl.pallas_call(
        _tc_xw_body,
        out_shape=jax.ShapeDtypeStruct((NPK, 16 * HID), f32),
    )(x.reshape(NPK_X, 16, D_FEAT), W_conv)

    xn_p, dinv_p = pl.pallas_call(
        _tc_xn_body,
        out_shape=(jax.ShapeDtypeStruct((NPK, 16 * HID), f32),
                   jax.ShapeDtypeStruct((NPK, 16 * HID), f32)),
    )(xw_p, deg_parts.reshape(NC, NPK, 16), bmat)

    agg_parts = pl.kernel(
        _sc_agg_body,
        out_type=jax.ShapeDtypeStruct((NC, NPAD, HID), f32),
        mesh=mesh,
        scratch_types=[
            pltpu.VMEM((ROWS_A, CHA), jnp.int32),
            pltpu.VMEM((ROWS_A, CHA), jnp.int32),
            [pltpu.VMEM((CHA, HID), f32) for _ in range(NBUF)],
            pltpu.VMEM_SHARED((NPAD, HID), f32),
            pltpu.VMEM_SHARED((NPAD, HID), f32),
            [pltpu.SemaphoreType.DMA for _ in range(NBUF)],
            [pltpu.SemaphoreType.DMA for _ in range(NBUF)],
        ],
        compiler_params=sc_params,
    )(eip, xn_p.reshape(NPAD, HID), zeros16)

    h_p = pl.pallas_call(
        _tc_h_body,
        out_shape=jax.ShapeDtypeStruct((NPK, 16 * HID), f32),
    )(agg_parts.reshape(NC, NPK, 16 * HID), xn_p, dinv_p,
      jnp.tile(b_conv.astype(f32), 16).reshape(1, 16 * HID))

    sums_parts, cnt_parts = pl.kernel(
        _sc_pool_body,
        out_type=(jax.ShapeDtypeStruct((NC, G_PAD, HID), f32),
                  jax.ShapeDtypeStruct((NC, G_PAD, HID), f32)),
        mesh=mesh,
        scratch_types=[
            pltpu.VMEM((POOL_CH,), jnp.int32),
            pltpu.VMEM((POOL_CH, HID), f32),
            pltpu.VMEM((POOL_CH, HID), f32),
            pltpu.VMEM_SHARED((G_PAD, HID), f32),
            pltpu.VMEM_SHARED((G_PAD, HID), f32),
            pltpu.SemaphoreType.DMA,
        ],
        compiler_params=sc_params,
    )(h_p.reshape(NPAD, HID), batch3, ones16, zerosg)

    woutr = W_out.transpose(1, 0, 2).reshape(HID * 4, N_MIC * N_ACTS)
    boutr = b_out.reshape(1, N_MIC * N_ACTS)

    outp = pl.pallas_call(
        _tc_head_body,
        out_shape=jax.ShapeDtypeStruct((G_PAD, N_MIC * N_ACTS), f32),
    )(sums_parts, cnt_parts, W1, b1.reshape(1, 64), W2, b2.reshape(1, 64),
      woutr, boutr)

    return outp[:N_GRAPHS].reshape(N_GRAPHS, N_MIC, N_ACTS)


# speedup vs baseline: 1.0164x; 1.0164x over previous
import jax
import jax.numpy as jnp
from jax import lax
from jax.experimental import pallas as pl
from jax.experimental.pallas import tpu as pltpu
from jax.experimental.pallas import tpu_sc as plsc

N_NODES = 10000
N_EDGES = 320000
N_GRAPHS = 100
D_FEAT = 128
HID = 16
N_MIC = 10
N_ACTS = 8

NC = 2
NS = 16
NW = NC * NS
CHA = 128
ROWS_A = 80
N_ROWS = NW * ROWS_A
E_PAD = N_ROWS * CHA
NPAD = 10240
NPK = NPAD // 16
NPK_X = N_NODES // 16
G_PAD = 128
POOL_CH = NPAD // NW

_HIGH = lax.Precision.HIGHEST
NBUF = 4


def _sc_deg_body(eip, zeros1, ones1, deg_out, idx_v, ones_v, sh_deg, dsem):
    c = lax.axis_index("c")
    s = lax.axis_index("s")

    @pl.when(s == 0)
    def _init():
        pltpu.sync_copy(zeros1, sh_deg)

    pltpu.sync_copy(ones1, ones_v)
    wid = s * NC + c
    pltpu.sync_copy(eip.at[1].at[pl.ds(wid * ROWS_A, ROWS_A)], idx_v)
    plsc.subcore_barrier()

    def group(g, carry):
        descs = [
            pltpu.async_copy(ones_v, sh_deg.at[idx_v.at[4 * g + b]], dsem,
                             add=True)
            for b in range(4)
        ]
        for d in descs:
            d.wait()
        return carry

    lax.fori_loop(0, ROWS_A // 4, group, 0)
    plsc.subcore_barrier()

    @pl.when(s == 0)
    def _flush():
        pltpu.sync_copy(sh_deg, deg_out.at[c])


def _sc_agg_body(eia, xn, zeros16, agg_out,
                 sidx, didx, bufs, sh_acc, xn_sh, gsems, ssems):
    c = lax.axis_index("c")
    s = lax.axis_index("s")

    @pl.when(s == 0)
    def _init():
        pltpu.sync_copy(zeros16, sh_acc)
        pltpu.sync_copy(xn, xn_sh)

    wid = s * NC + c
    pltpu.sync_copy(eia.at[0].at[pl.ds(wid * ROWS_A, ROWS_A)], sidx)
    pltpu.sync_copy(eia.at[1].at[pl.ds(wid * ROWS_A, ROWS_A)], didx)
    plsc.subcore_barrier()

    for b in range(NBUF):
        pltpu.async_copy(xn_sh.at[sidx.at[b]], bufs[b], gsems[b])

    def step(t, carry):
        j = NBUF * t
        for b in range(NBUF):
            pltpu.make_async_copy(xn_sh.at[sidx.at[j + b]], bufs[b],
                                  gsems[b]).wait()
            pltpu.async_copy(bufs[b], sh_acc.at[didx.at[j + b]], ssems[b],
                             add=True)
        for b in range(NBUF):
            @pl.when(j + b + NBUF < ROWS_A)
            def _next(b=b):
                pltpu.make_async_copy(bufs[b], sh_acc.at[didx.at[j + b]],
                                      ssems[b]).wait()
                pltpu.async_copy(xn_sh.at[sidx.at[j + b + NBUF]], bufs[b],
                                 gsems[b])
        return carry

    lax.fori_loop(0, ROWS_A // NBUF, step, 0)
    for b in range(NBUF):
        pltpu.make_async_copy(bufs[b],
                              sh_acc.at[didx.at[ROWS_A - NBUF + b]],
                              ssems[b]).wait()
    plsc.subcore_barrier()

    @pl.when(s == 0)
    def _flush():
        pltpu.sync_copy(sh_acc, agg_out.at[c])


def _sc_pool_body(h, batch3, ones16, zerosg, sums_out, cnt_out,
                  bidx, h_v, ones_v, sh_sums, sh_cnt, psem):
    c = lax.axis_index("c")
    s = lax.axis_index("s")

    @pl.when(s == 0)
    def _init():
        pltpu.sync_copy(zerosg, sh_sums)
        pltpu.sync_copy(zerosg, sh_cnt)

    wid = s * NC + c
    pltpu.sync_copy(batch3.at[wid], bidx)
    pltpu.sync_copy(h.at[pl.ds(wid * POOL_CH, POOL_CH)], h_v)
    pltpu.sync_copy(ones16, ones_v)
    plsc.subcore_barrier()

    d1 = pltpu.async_copy(h_v, sh_sums.at[bidx], psem, add=True)
    d2 = pltpu.async_copy(ones_v, sh_cnt.at[bidx], psem, add=True)
    d1.wait()
    d2.wait()
    plsc.subcore_barrier()

    @pl.when(s == 0)
    def _flush():
        pltpu.sync_copy(sh_sums, sums_out.at[c])
        pltpu.sync_copy(sh_cnt, cnt_out.at[c])


def _tc_xw_body(x3_ref, w_ref, xw_ref):
    cols = []
    for k in range(16):
        xk = x3_ref[:, k, :]
        cols.append(jnp.dot(xk, w_ref[...],
                            preferred_element_type=jnp.float32,
                            precision=_HIGH))
    xw_p = jnp.concatenate(cols, axis=1)
    xw_ref[...] = jnp.concatenate(
        [xw_p, jnp.zeros((NPK - NPK_X, 16 * HID), jnp.float32)], axis=0)


def _tc_xn_body(xw_ref, degp_ref, bmat_ref, xn_ref, dinv_ref):
    deg2 = degp_ref[0] + degp_ref[1]
    deg_p = jnp.dot(deg2, bmat_ref[...],
                    preferred_element_type=jnp.float32,
                    precision=_HIGH) + 1.0
    dinv_p = lax.rsqrt(deg_p)
    dinv_ref[...] = dinv_p
    xn_ref[...] = xw_ref[...] * dinv_p


def _tc_h_body(aggp_ref, xn_ref, dinv_ref, bconv_ref, h_ref):
    agg = aggp_ref[0] + aggp_ref[1] + xn_ref[...]
    h_ref[...] = jnp.maximum(dinv_ref[...] * agg + bconv_ref[...], 0.0)


def _elu(v):
    return jnp.where(v > 0.0, v, jnp.exp(jnp.minimum(v, 0.0)) - 1.0)


def _tc_head_body(sums_ref, cnt_ref, w1_ref, b1_ref, w2_ref, b2_ref,
                  wout_ref, bout_ref, out_ref):
    sums = sums_ref[0] + sums_ref[1]
    cnt = cnt_ref[0] + cnt_ref[1]
    g = sums / jnp.maximum(cnt, 1.0)
    g = _elu(jnp.dot(g, w1_ref[...],
                     preferred_element_type=jnp.float32, precision=_HIGH)
             + b1_ref[...])
    g = _elu(jnp.dot(g, w2_ref[...],
                     preferred_element_type=jnp.float32, precision=_HIGH)
             + b2_ref[...])
    out_ref[...] = jnp.dot(g, wout_ref[...],
                           preferred_element_type=jnp.float32,
                           precision=_HIGH) + bout_ref[...]


def kernel(x, edge_index, batch, W_conv, b_conv, W1, b1, W2, b2, W_out, b_out):
    f32 = jnp.float32
    eip = jnp.pad(edge_index.astype(jnp.int32),
                  ((0, 0), (0, E_PAD - N_EDGES)),
                  constant_values=N_NODES).reshape(2, N_ROWS, CHA)
    zeros16 = jnp.zeros((NPAD, HID), f32)
    zeros1 = jnp.zeros((NPAD,), f32)
    ones1 = jnp.ones((CHA,), f32)
    ones16 = jnp.ones((POOL_CH, HID), f32)
    zerosg = jnp.zeros((G_PAD, HID), f32)
    bmat = jnp.repeat(jnp.eye(16, dtype=f32), 16, axis=1)
    batch3 = jnp.pad(batch.astype(jnp.int32), (0, NPAD - N_NODES),
                     constant_values=G_PAD - 1).reshape(NW, POOL_CH)

    mesh = plsc.VectorSubcoreMesh(core_axis_name="c", subcore_axis_name="s",
                                  num_cores=NC, num_subcores=NS)
    sc_params = pltpu.CompilerParams(use_tc_tiling_on_sc=False)

    deg_parts = pl.kernel(
        _sc_deg_body,
        out_type=jax.ShapeDtypeStruct((NC, NPAD), f32),
        mesh=mesh,
        scratch_types=[
            pltpu.VMEM((ROWS_A, CHA), jnp.int32),
            pltpu.VMEM((CHA,), f32),
            pltpu.VMEM_SHARED((NPAD,), f32),
            pltpu.SemaphoreType.DMA,
        ],
        compiler_params=sc_params,
    )(eip, zeros1, ones1)

    xw_p = pl.pallas_call(
        _tc_xw_body,
        out_shape=jax.ShapeDtypeStruct((NPK, 16 * HID), f32),
    )(x.reshape(NPK_X, 16, D_FEAT), W_conv)

    xn_p, dinv_p = pl.pallas_call(
        _tc_xn_body,
        out_shape=(jax.ShapeDtypeStruct((NPK, 16 * HID), f32),
                   jax.ShapeDtypeStruct((NPK, 16 * HID), f32)),
    )(xw_p, deg_parts.reshape(NC, NPK, 16), bmat)

    agg_parts = pl.kernel(
        _sc_agg_body,
        out_type=jax.ShapeDtypeStruct((NC, NPAD, HID), f32),
        mesh=mesh,
        scratch_types=[
            pltpu.VMEM((ROWS_A, CHA), jnp.int32),
            pltpu.VMEM((ROWS_A, CHA), jnp.int32),
            [pltpu.VMEM((CHA, HID), f32) for _ in range(NBUF)],
            pltpu.VMEM_SHARED((NPAD, HID), f32),
            pltpu.VMEM_SHARED((NPAD, HID), f32),
            [pltpu.SemaphoreType.DMA for _ in range(NBUF)],
            [pltpu.SemaphoreType.DMA for _ in range(NBUF)],
        ],
        compiler_params=sc_params,
    )(eip, xn_p.reshape(NPAD, HID), zeros16)

    h_p = pl.pallas_call(
        _tc_h_body,
        out_shape=jax.ShapeDtypeStruct((NPK, 16 * HID), f32),
    )(agg_parts.reshape(NC, NPK, 16 * HID), xn_p, dinv_p,
      jnp.tile(b_conv.astype(f32), 16).reshape(1, 16 * HID))

    sums_parts, cnt_parts = pl.kernel(
        _sc_pool_body,
        out_type=(jax.ShapeDtypeStruct((NC, G_PAD, HID), f32),
                  jax.ShapeDtypeStruct((NC, G_PAD, HID), f32)),
        mesh=mesh,
        scratch_types=[
            pltpu.VMEM((POOL_CH,), jnp.int32),
            pltpu.VMEM((POOL_CH, HID), f32),
            pltpu.VMEM((POOL_CH, HID), f32),
            pltpu.VMEM_SHARED((G_PAD, HID), f32),
            pltpu.VMEM_SHARED((G_PAD, HID), f32),
            pltpu.SemaphoreType.DMA,
        ],
        compiler_params=sc_params,
    )(h_p.reshape(NPAD, HID), batch3, ones16, zerosg)

    woutr = W_out.transpose(1, 0, 2).reshape(HID * 4, N_MIC * N_ACTS)
    boutr = b_out.reshape(1, N_MIC * N_ACTS)

    outp = pl.pallas_call(
        _tc_head_body,
        out_shape=jax.ShapeDtypeStruct((G_PAD, N_MIC * N_ACTS), f32),
    )(sums_parts, cnt_parts, W1, b1.reshape(1, 64), W2, b2.reshape(1, 64),
      woutr, boutr)

    return outp[:N_GRAPHS].reshape(N_GRAPHS, N_MIC, N_ACTS)
